# initial kernel scaffold (unmeasured)
import jax
import jax.numpy as jnp
from jax import lax
from jax.experimental import pallas as pl
from jax.experimental.pallas import tpu as pltpu

N_DEV = 16


def kernel(x, router_W, route_idx, expert_W, shared_W):
    n_tok, d = x.shape
    e_loc, _, h_dim = expert_W.shape

    def body(x_ref, rw_ref, idx_ref, ew_ref, sw_ref, out_ref,
             comm_ref, send_sems, recv_sems, credit_sem):
        my = lax.axis_index("i")
        left = (my + N_DEV - 1) % N_DEV
        right = (my + 1) % N_DEV

        barrier = pltpu.get_barrier_semaphore()
        for nbr in (left, right):
            pl.semaphore_signal(
                barrier, inc=1,
                device_id=(nbr,), device_id_type=pl.DeviceIdType.MESH,
            )
        pl.semaphore_wait(barrier, 2)

        xv = x_ref[...]

        scores = jnp.dot(xv, rw_ref[...], preferred_element_type=jnp.float32)
        s_max = jnp.max(scores, axis=-1, keepdims=True)
        ex = jnp.exp(scores - s_max)
        probs = ex / jnp.sum(ex, axis=-1, keepdims=True)
        idx = idx_ref[...]
        eid = lax.broadcasted_iota(jnp.int32, scores.shape, 1)
        p = jnp.sum(jnp.where(eid == idx, probs, 0.0), axis=-1,
                    keepdims=True)

        out_ref[...] = jnp.dot(xv, sw_ref[...],
                               preferred_element_type=jnp.float32)

        for hop in range(N_DEV):
            slot = hop % 2
            nslot = (hop + 1) % 2
            if hop < N_DEV - 1:
                if hop >= 1:
                    pl.semaphore_wait(credit_sem, 1)
                rdma = pltpu.make_async_remote_copy(
                    src_ref=(ew_ref if hop == 0 else comm_ref.at[slot]),
                    dst_ref=comm_ref.at[nslot],
                    send_sem=send_sems.at[slot],
                    recv_sem=recv_sems.at[nslot],
                    device_id=(right,),
                    device_id_type=pl.DeviceIdType.MESH,
                )
                rdma.start()

            owner = (my + N_DEV - hop) % N_DEV
            base = owner * e_loc
            for e in range(e_loc):
                w_e = ew_ref[e] if hop == 0 else comm_ref[slot, e]
                m = jnp.where(idx == base + e, p, 0.0)
                out_ref[...] += jnp.dot(xv * m, w_e,
                                        preferred_element_type=jnp.float32)

            if hop < N_DEV - 1:
                rdma.wait()
                if hop < N_DEV - 2:
                    pl.semaphore_signal(
                        credit_sem, inc=1,
                        device_id=(left,),
                        device_id_type=pl.DeviceIdType.MESH,
                    )

    return pl.pallas_call(
        body,
        out_shape=jax.ShapeDtypeStruct((n_tok, h_dim), jnp.float32),
        in_specs=[pl.BlockSpec(memory_space=pltpu.VMEM)] * 5,
        out_specs=pl.BlockSpec(memory_space=pltpu.VMEM),
        scratch_shapes=[
            pltpu.VMEM((2, e_loc, d, h_dim), jnp.float32),
            pltpu.SemaphoreType.DMA((2,)),
            pltpu.SemaphoreType.DMA((2,)),
            pltpu.SemaphoreType.REGULAR,
        ],
        compiler_params=pltpu.CompilerParams(collective_id=0),
    )(x, router_W, route_idx, expert_W, shared_W)


# baseline (device time: 1461574 ns/iter reference)
import jax
import jax.numpy as jnp
from jax import lax
from jax.experimental import pallas as pl
from jax.experimental.pallas import tpu as pltpu

N_DEV = 16


def kernel(x, router_W, route_idx, expert_W, shared_W):
    n_tok, d = x.shape
    e_loc, _, h_dim = expert_W.shape

    def body(x_ref, rw_ref, idx_ref, ew_ref, sw_ref, out_ref,
             comm_ref, send_sems, recv_sems, credit_sem):
        my = lax.axis_index("i")
        left = (my + N_DEV - 1) % N_DEV
        right = (my + 1) % N_DEV

        barrier = pltpu.get_barrier_semaphore()
        for nbr in (left, right):
            pl.semaphore_signal(
                barrier, inc=1,
                device_id=(nbr,), device_id_type=pl.DeviceIdType.MESH,
            )
        pl.semaphore_wait(barrier, 2)

        xv = x_ref[...]

        scores = jnp.dot(xv, rw_ref[...], preferred_element_type=jnp.float32)
        s_max = jnp.max(scores, axis=-1, keepdims=True)
        ex = jnp.exp(scores - s_max)
        probs = ex / jnp.sum(ex, axis=-1, keepdims=True)
        idx = idx_ref[...]
        eid = lax.broadcasted_iota(jnp.int32, scores.shape, 1)
        p = jnp.sum(jnp.where(eid == idx, probs, 0.0), axis=-1,
                    keepdims=True)

        out_ref[...] = jnp.dot(xv, sw_ref[...],
                               preferred_element_type=jnp.float32)

        comm_ref[0] = ew_ref[...]

        def forward_rdma(slot, nslot):
            return pltpu.make_async_remote_copy(
                src_ref=comm_ref.at[slot],
                dst_ref=comm_ref.at[nslot],
                send_sem=send_sems.at[slot],
                recv_sem=recv_sems.at[nslot],
                device_id=(right,),
                device_id_type=pl.DeviceIdType.MESH,
            )

        def hop_body(hop, carry):
            slot = jnp.remainder(hop, 2)
            nslot = 1 - slot

            @pl.when(jnp.logical_and(hop >= 1, hop < N_DEV - 1))
            def _():
                pl.semaphore_wait(credit_sem, 1)

            @pl.when(hop < N_DEV - 1)
            def _():
                forward_rdma(slot, nslot).start()

            owner = jnp.remainder(my + N_DEV - hop, N_DEV)
            base = owner * e_loc

            def expert_body(e, c):
                w_e = comm_ref[slot, e]
                m = jnp.where(idx == base + e, p, 0.0)
                out_ref[...] += jnp.dot(xv * m, w_e,
                                        preferred_element_type=jnp.float32)
                return c

            lax.fori_loop(0, e_loc, expert_body, 0)

            @pl.when(hop < N_DEV - 1)
            def _():
                forward_rdma(slot, nslot).wait()

            @pl.when(hop < N_DEV - 2)
            def _():
                pl.semaphore_signal(
                    credit_sem, inc=1,
                    device_id=(left,),
                    device_id_type=pl.DeviceIdType.MESH,
                )

            return carry

        lax.fori_loop(0, N_DEV, hop_body, 0)

    return pl.pallas_call(
        body,
        out_shape=jax.ShapeDtypeStruct((n_tok, h_dim), jnp.float32),
        in_specs=[pl.BlockSpec(memory_space=pltpu.VMEM)] * 5,
        out_specs=pl.BlockSpec(memory_space=pltpu.VMEM),
        scratch_shapes=[
            pltpu.VMEM((2, e_loc, d, h_dim), jnp.float32),
            pltpu.SemaphoreType.DMA((2,)),
            pltpu.SemaphoreType.DMA((2,)),
            pltpu.SemaphoreType.REGULAR,
        ],
        compiler_params=pltpu.CompilerParams(
            collective_id=0, vmem_limit_bytes=96 * 1024 * 1024
        ),
    )(x, router_W, route_idx, expert_W, shared_W)


# device time: 786897 ns/iter; 1.8574x vs baseline; 1.8574x over previous
import jax
import jax.numpy as jnp
from jax import lax
from jax.experimental import pallas as pl
from jax.experimental.pallas import tpu as pltpu

N_DEV = 16


def kernel(x, router_W, route_idx, expert_W, shared_W):
    n_tok, d = x.shape
    e_loc, _, h_dim = expert_W.shape

    def body(x_ref, rw_ref, idx_ref, ew_ref, sw_ref, out_ref,
             comm_ref, send_sems, recv_sems, credit_sem):
        my = lax.axis_index("i")
        left = (my + N_DEV - 1) % N_DEV
        right = (my + 1) % N_DEV

        barrier = pltpu.get_barrier_semaphore()
        for nbr in (left, right):
            pl.semaphore_signal(
                barrier, inc=1,
                device_id=(nbr,), device_id_type=pl.DeviceIdType.MESH,
            )
        pl.semaphore_wait(barrier, 2)

        xv = x_ref[...]

        scores = jnp.dot(xv, rw_ref[...], preferred_element_type=jnp.float32)
        s_max = jnp.max(scores, axis=-1, keepdims=True)
        ex = jnp.exp(scores - s_max)
        probs = ex / jnp.sum(ex, axis=-1, keepdims=True)
        idx = idx_ref[...]
        eid = lax.broadcasted_iota(jnp.int32, scores.shape, 1)
        p = jnp.sum(jnp.where(eid == idx, probs, 0.0), axis=-1,
                    keepdims=True)

        out_ref[...] = jnp.dot(xv, sw_ref[...],
                               preferred_element_type=jnp.float32)

        comm_ref[0] = ew_ref[...].astype(jnp.bfloat16)

        def forward_rdma(slot, nslot):
            return pltpu.make_async_remote_copy(
                src_ref=comm_ref.at[slot],
                dst_ref=comm_ref.at[nslot],
                send_sem=send_sems.at[slot],
                recv_sem=recv_sems.at[nslot],
                device_id=(right,),
                device_id_type=pl.DeviceIdType.MESH,
            )

        def hop_body(hop, carry):
            slot = jnp.remainder(hop, 2)
            nslot = 1 - slot

            @pl.when(jnp.logical_and(hop >= 1, hop < N_DEV - 1))
            def _():
                pl.semaphore_wait(credit_sem, 1)

            @pl.when(hop < N_DEV - 1)
            def _():
                forward_rdma(slot, nslot).start()

            owner = jnp.remainder(my + N_DEV - hop, N_DEV)
            base = owner * e_loc

            def expert_body(e, c):
                w_e = comm_ref[slot, e]
                m = jnp.where(idx == base + e, p, 0.0)
                xm = (xv * m).astype(jnp.bfloat16)
                out_ref[...] += jnp.dot(xm, w_e,
                                        preferred_element_type=jnp.float32)
                return c

            lax.fori_loop(0, e_loc, expert_body, 0)

            @pl.when(hop < N_DEV - 1)
            def _():
                forward_rdma(slot, nslot).wait()

            @pl.when(hop < N_DEV - 2)
            def _():
                pl.semaphore_signal(
                    credit_sem, inc=1,
                    device_id=(left,),
                    device_id_type=pl.DeviceIdType.MESH,
                )

            return carry

        lax.fori_loop(0, N_DEV, hop_body, 0)

    return pl.pallas_call(
        body,
        out_shape=jax.ShapeDtypeStruct((n_tok, h_dim), jnp.float32),
        in_specs=[pl.BlockSpec(memory_space=pltpu.VMEM)] * 5,
        out_specs=pl.BlockSpec(memory_space=pltpu.VMEM),
        scratch_shapes=[
            pltpu.VMEM((2, e_loc, d, h_dim), jnp.bfloat16),
            pltpu.SemaphoreType.DMA((2,)),
            pltpu.SemaphoreType.DMA((2,)),
            pltpu.SemaphoreType.REGULAR,
        ],
        compiler_params=pltpu.CompilerParams(
            collective_id=0, vmem_limit_bytes=96 * 1024 * 1024
        ),
    )(x, router_W, route_idx, expert_W, shared_W)


# device time: 435337 ns/iter; 3.3573x vs baseline; 1.8076x over previous
import jax
import jax.numpy as jnp
from jax import lax
from jax.experimental import pallas as pl
from jax.experimental.pallas import tpu as pltpu

N_DEV = 16


def kernel(x, router_W, route_idx, expert_W, shared_W):
    n_tok, d = x.shape
    e_loc, _, h_dim = expert_W.shape

    def body(x_ref, rw_ref, idx_ref, ew_ref, sw_ref, out_ref,
             comm_r, send_r, recv_r, credit_r,
             comm_l, send_l, recv_l, credit_l):
        my = lax.axis_index("i")
        left = (my + N_DEV - 1) % N_DEV
        right = (my + 1) % N_DEV

        barrier = pltpu.get_barrier_semaphore()
        for nbr in (left, right):
            pl.semaphore_signal(
                barrier, inc=1,
                device_id=(nbr,), device_id_type=pl.DeviceIdType.MESH,
            )
        pl.semaphore_wait(barrier, 2)

        xv = x_ref[...]

        scores = jnp.dot(xv, rw_ref[...], preferred_element_type=jnp.float32)
        s_max = jnp.max(scores, axis=-1, keepdims=True)
        ex = jnp.exp(scores - s_max)
        probs = ex / jnp.sum(ex, axis=-1, keepdims=True)
        idx = idx_ref[...]
        eid = lax.broadcasted_iota(jnp.int32, scores.shape, 1)
        p = jnp.sum(jnp.where(eid == idx, probs, 0.0), axis=-1,
                    keepdims=True)

        out_ref[...] = jnp.dot(xv, sw_ref[...],
                               preferred_element_type=jnp.float32)

        own_b = ew_ref[...].astype(jnp.bfloat16)
        comm_r[0] = own_b
        comm_l[0] = own_b

        HOPS_R = 8
        HOPS_L = 7

        def ring_rdma(comm, send_sems, recv_sems, slot, nslot, dst):
            return pltpu.make_async_remote_copy(
                src_ref=comm.at[slot],
                dst_ref=comm.at[nslot],
                send_sem=send_sems.at[slot],
                recv_sem=recv_sems.at[nslot],
                device_id=(dst,),
                device_id_type=pl.DeviceIdType.MESH,
            )

        def block_compute(comm, slot, owner):
            base = owner * e_loc

            def expert_body(e, c):
                w_e = comm[slot, e]
                m = jnp.where(idx == base + e, p, 0.0)
                xm = (xv * m).astype(jnp.bfloat16)
                out_ref[...] += jnp.dot(xm, w_e,
                                        preferred_element_type=jnp.float32)
                return c

            lax.fori_loop(0, e_loc, expert_body, 0)

        def hop_body(hop, carry):
            slot = jnp.remainder(hop, 2)
            nslot = 1 - slot

            @pl.when(jnp.logical_and(hop >= 1, hop < HOPS_R))
            def _():
                pl.semaphore_wait(credit_r, 1)

            @pl.when(hop < HOPS_R)
            def _():
                ring_rdma(comm_r, send_r, recv_r, slot, nslot, right).start()

            @pl.when(jnp.logical_and(hop >= 1, hop < HOPS_L))
            def _():
                pl.semaphore_wait(credit_l, 1)

            @pl.when(hop < HOPS_L)
            def _():
                ring_rdma(comm_l, send_l, recv_l, slot, nslot, left).start()

            block_compute(comm_r, slot, jnp.remainder(my + N_DEV - hop, N_DEV))

            @pl.when(jnp.logical_and(hop >= 1, hop <= HOPS_L))
            def _():
                block_compute(comm_l, slot, jnp.remainder(my + hop, N_DEV))

            @pl.when(hop < HOPS_R)
            def _():
                ring_rdma(comm_r, send_r, recv_r, slot, nslot, right).wait()

            @pl.when(hop < HOPS_R - 1)
            def _():
                pl.semaphore_signal(
                    credit_r, inc=1,
                    device_id=(left,), device_id_type=pl.DeviceIdType.MESH,
                )

            @pl.when(hop < HOPS_L)
            def _():
                ring_rdma(comm_l, send_l, recv_l, slot, nslot, left).wait()

            @pl.when(hop < HOPS_L - 1)
            def _():
                pl.semaphore_signal(
                    credit_l, inc=1,
                    device_id=(right,), device_id_type=pl.DeviceIdType.MESH,
                )

            return carry

        lax.fori_loop(0, HOPS_R + 1, hop_body, 0)

    return pl.pallas_call(
        body,
        out_shape=jax.ShapeDtypeStruct((n_tok, h_dim), jnp.float32),
        in_specs=[pl.BlockSpec(memory_space=pltpu.VMEM)] * 5,
        out_specs=pl.BlockSpec(memory_space=pltpu.VMEM),
        scratch_shapes=[
            pltpu.VMEM((2, e_loc, d, h_dim), jnp.bfloat16),
            pltpu.SemaphoreType.DMA((2,)),
            pltpu.SemaphoreType.DMA((2,)),
            pltpu.SemaphoreType.REGULAR,
            pltpu.VMEM((2, e_loc, d, h_dim), jnp.bfloat16),
            pltpu.SemaphoreType.DMA((2,)),
            pltpu.SemaphoreType.DMA((2,)),
            pltpu.SemaphoreType.REGULAR,
        ],
        compiler_params=pltpu.CompilerParams(
            collective_id=0, vmem_limit_bytes=96 * 1024 * 1024
        ),
    )(x, router_W, route_idx, expert_W, shared_W)
